# baseline (speedup 1.0000x reference)
import functools

import jax
import jax.numpy as jnp
from jax import lax
from jax.experimental import pallas as pl
from jax.experimental.pallas import tpu as pltpu
from jax.experimental.pallas import tpu_sc as plsc

NU = 50000
NPR = 50000
EDG = 500000
H = 128
NCHUNK = 8
CPS = NCHUNK // 2
EB = 2000
NEB = EDG // EB
RB = 2000
NRB = NU // RB
BN = 2000
_f32 = jnp.float32

_mesh = plsc.VectorSubcoreMesh(core_axis_name="c", subcore_axis_name="s")
_sc_params = pltpu.CompilerParams(use_tc_tiling_on_sc=False)


@functools.partial(
    pl.kernel,
    out_type=[jax.ShapeDtypeStruct((NU,), _f32),
              jax.ShapeDtypeStruct((NPR,), _f32)],
    mesh=_mesh,
    scratch_types=[
        pltpu.VMEM((EB,), jnp.int32),
        pltpu.VMEM((EB,), _f32),
        pltpu.VMEM((RB,), _f32),
        pltpu.VMEM_SHARED((NU,), _f32),
    ],
    compiler_params=_sc_params,
)
def _sc_counts(src_hbm, dst_hbm, inv_u, inv_p, idx_v, ones_v, buf_v, cnt_sh):
    s = lax.axis_index("s")
    c = lax.axis_index("c")

    for i in range(EB // 16):
        ones_v[pl.ds(i * 16, 16)] = jnp.full((16,), 1.0, _f32)
    for i in range(RB // 16):
        buf_v[pl.ds(i * 16, 16)] = jnp.zeros((16,), _f32)

    def run(idx_hbm, out_hbm, cnt_s):
        for k in range(2):
            j = s + k * 16

            @pl.when(j < NRB)
            def _():
                pltpu.sync_copy(buf_v, cnt_s.at[pl.ds(j * RB, RB)])
        plsc.subcore_barrier()
        for k in range(16):
            j = s + k * 16

            @pl.when(j < NEB)
            def _():
                pltpu.sync_copy(idx_hbm.at[pl.ds(j * EB, EB)], idx_v)
                pltpu.sync_copy(ones_v, cnt_s.at[idx_v], add=True)
        plsc.subcore_barrier()
        for k in range(2):
            j = s + k * 16

            @pl.when(j < NRB)
            def _():
                pltpu.sync_copy(cnt_s.at[pl.ds(j * RB, RB)], buf_v)
                for i in range(RB // 16):
                    v = buf_v[pl.ds(i * 16, 16)]
                    buf_v[pl.ds(i * 16, 16)] = 1.0 / jnp.maximum(v, 1.0)
                pltpu.sync_copy(buf_v, out_hbm.at[pl.ds(j * RB, RB)])

    @pl.when(c == 0)
    def _():
        run(dst_hbm, inv_p, cnt_sh)

    @pl.when(c == 1)
    def _():
        run(src_hbm, inv_u, cnt_sh)


@functools.partial(
    pl.kernel,
    out_type=jax.ShapeDtypeStruct((NPR * NCHUNK, 16), _f32),
    mesh=_mesh,
    scratch_types=[
        pltpu.VMEM((EB,), jnp.int32),
        pltpu.VMEM((EB,), jnp.int32),
        pltpu.VMEM((EB, 16), _f32),
        pltpu.VMEM((RB, 16), _f32),
        pltpu.VMEM((RB,), jnp.int32),
        pltpu.VMEM_SHARED((NPR, 16), _f32),
        pltpu.SemaphoreType.DMA,
    ],
    compiler_params=_sc_params,
)
def _sc_agg_dir(hf, gat8_hbm, sct_hbm, agg_out,
                ig_v, is_v, rw_v, zb_v, iot_v, acc, sem):
    s = lax.axis_index("s")
    c = lax.axis_index("c")
    VSZ = NU * NCHUNK - (NCHUNK - 1)

    def zrow(i, _):
        zb_v[i] = jnp.zeros((16,), _f32)
        return 0

    lax.fori_loop(0, RB, zrow, 0)

    def irow(i, _):
        iot_v[pl.ds(i * 16, 16)] = (lax.iota(jnp.int32, 16) + i * 16) * NCHUNK
        return 0

    lax.fori_loop(0, RB // 16, irow, 0)

    def one_pass(chunk, hf, gat8_hbm, sct_hbm, out_hbm):
        for k in range(2):
            j = s + k * 16

            @pl.when(j < NRB)
            def _():
                pltpu.sync_copy(zb_v, acc.at[pl.ds(j * RB, RB)])
        plsc.subcore_barrier()
        hview = hf.at[pl.ds(chunk, VSZ)]
        for k in range(16):
            j = s + k * 16

            @pl.when(j < NEB)
            def _():
                pltpu.sync_copy(gat8_hbm.at[pl.ds(j * EB, EB)], ig_v)
                pltpu.sync_copy(sct_hbm.at[pl.ds(j * EB, EB)], is_v)
                pltpu.async_copy(hview.at[ig_v], rw_v, sem).wait()
                pltpu.sync_copy(rw_v, acc.at[is_v], add=True)
        plsc.subcore_barrier()
        oview = out_hbm.at[pl.ds(chunk, VSZ)]
        for k in range(2):
            j = s + k * 16

            @pl.when(j < NRB)
            def _():
                base = j * (RB * NCHUNK)

                def wrow(i, _):
                    ig_v[pl.ds(i * 16, 16)] = iot_v[pl.ds(i * 16, 16)] + base
                    return 0

                lax.fori_loop(0, RB // 16, wrow, 0)
                pltpu.sync_copy(acc.at[pl.ds(j * RB, RB)], rw_v)
                pltpu.sync_copy(rw_v, oview.at[ig_v])
        plsc.subcore_barrier()

    for cc in range(CPS):
        chunk = c * CPS + cc
        one_pass(chunk, hf, gat8_hbm, sct_hbm, agg_out)


def _lin_body(x_ref, w_ref, b_ref, o_ref):
    o_ref[...] = lax.dot_general(
        x_ref[...], w_ref[...], (((1,), (1,)), ((), ())),
        preferred_element_type=_f32) + b_ref[...]


def _linear(x, w, b):
    n, fi = x.shape
    fo = w.shape[0]
    return pl.pallas_call(
        _lin_body,
        grid=(n // BN,),
        in_specs=[pl.BlockSpec((BN, fi), lambda i: (i, 0)),
                  pl.BlockSpec((fo, fi), lambda i: (0, 0)),
                  pl.BlockSpec((1, fo), lambda i: (0, 0))],
        out_specs=pl.BlockSpec((BN, fo), lambda i: (i, 0)),
        out_shape=jax.ShapeDtypeStruct((n, fo), _f32),
    )(x, w, b.reshape(1, fo))


def _upd_body(agg_ref, inv_ref, h_ref, wl_ref, bl_ref, wr_ref, o_ref):
    a = agg_ref[...] * inv_ref[...]
    t = lax.dot_general(a, wl_ref[...], (((1,), (1,)), ((), ())),
                        preferred_element_type=_f32)
    t = t + lax.dot_general(h_ref[...], wr_ref[...], (((1,), (1,)), ((), ())),
                            preferred_element_type=_f32)
    o_ref[...] = jnp.maximum(t + bl_ref[...], 0.0)


def _update(agg, inv, h, wl, bl, wr):
    n = h.shape[0]
    return pl.pallas_call(
        _upd_body,
        grid=(n // BN,),
        in_specs=[pl.BlockSpec((BN, H), lambda i: (i, 0)),
                  pl.BlockSpec((BN, 1), lambda i: (i, 0)),
                  pl.BlockSpec((BN, H), lambda i: (i, 0)),
                  pl.BlockSpec((H, H), lambda i: (0, 0)),
                  pl.BlockSpec((1, H), lambda i: (0, 0)),
                  pl.BlockSpec((H, H), lambda i: (0, 0))],
        out_specs=pl.BlockSpec((BN, H), lambda i: (i, 0)),
        out_shape=jax.ShapeDtypeStruct((n, H), _f32),
    )(agg, inv.reshape(n, 1), h, wl, bl.reshape(1, H), wr)


def kernel(x_user, x_problem, edge_index, user_W, user_b, problem_W, problem_b,
           l0_u2p_Wl, l0_u2p_bl, l0_u2p_Wr, l0_p2u_Wl, l0_p2u_bl, l0_p2u_Wr,
           l1_u2p_Wl, l1_u2p_bl, l1_u2p_Wr, l1_p2u_Wl, l1_p2u_bl, l1_p2u_Wr,
           out_user_W, out_user_b, out_problem_W, out_problem_b):
    src = edge_index[0].astype(jnp.int32)
    dst = edge_index[1].astype(jnp.int32)
    src8 = src * NCHUNK
    dst8 = dst * NCHUNK

    inv_u, inv_p = _sc_counts(src, dst)

    hu = _linear(x_user, user_W, user_b)
    hp = _linear(x_problem, problem_W, problem_b)

    ws = tuple(jnp.stack([a, b]) for a, b in
               ((l0_u2p_Wl, l1_u2p_Wl), (l0_u2p_bl, l1_u2p_bl),
                (l0_u2p_Wr, l1_u2p_Wr), (l0_p2u_Wl, l1_p2u_Wl),
                (l0_p2u_bl, l1_p2u_bl), (l0_p2u_Wr, l1_p2u_Wr)))

    def step(carry, w):
        hu, hp = carry
        Wl_p, bl_p, Wr_p, Wl_u, bl_u, Wr_u = w
        aggPf = _sc_agg_dir(hu.reshape(NU * NCHUNK, 16), src8, dst)
        hp_new = _update(aggPf.reshape(NPR, H), inv_p, hp, Wl_p, bl_p, Wr_p)
        aggUf = _sc_agg_dir(hp.reshape(NPR * NCHUNK, 16), dst8, src)
        hu_new = _update(aggUf.reshape(NU, H), inv_u, hu, Wl_u, bl_u, Wr_u)
        return (hu_new, hp_new), None

    (hu, hp), _ = lax.scan(step, (hu, hp), ws)

    out_u = _linear(hu, out_user_W, out_user_b)
    out_p = _linear(hp, out_problem_W, out_problem_b)
    return (out_u, out_p)


# speedup vs baseline: 1.1281x; 1.1281x over previous
import functools

import jax
import jax.numpy as jnp
from jax import lax
from jax.experimental import pallas as pl
from jax.experimental.pallas import tpu as pltpu
from jax.experimental.pallas import tpu_sc as plsc

NU = 50000
NPR = 50000
EDG = 500000
H = 128
NCHUNK = 8
CPS = NCHUNK // 2
EB = 2000
NEB = EDG // EB
RB = 2000
NRB = NU // RB
BN = 2000
_f32 = jnp.float32

_mesh = plsc.VectorSubcoreMesh(core_axis_name="c", subcore_axis_name="s")
_sc_params = pltpu.CompilerParams(use_tc_tiling_on_sc=False)


@functools.partial(
    pl.kernel,
    out_type=[jax.ShapeDtypeStruct((NU,), _f32),
              jax.ShapeDtypeStruct((NPR,), _f32)],
    mesh=_mesh,
    scratch_types=[
        pltpu.VMEM((EB,), jnp.int32),
        pltpu.VMEM((EB,), _f32),
        pltpu.VMEM((RB,), _f32),
        pltpu.VMEM_SHARED((NU,), _f32),
    ],
    compiler_params=_sc_params,
)
def _sc_counts(src_hbm, dst_hbm, inv_u, inv_p, idx_v, ones_v, buf_v, cnt_sh):
    s = lax.axis_index("s")
    c = lax.axis_index("c")

    for i in range(EB // 16):
        ones_v[pl.ds(i * 16, 16)] = jnp.full((16,), 1.0, _f32)
    for i in range(RB // 16):
        buf_v[pl.ds(i * 16, 16)] = jnp.zeros((16,), _f32)

    def run(idx_hbm, out_hbm, cnt_s):
        for k in range(2):
            j = s + k * 16

            @pl.when(j < NRB)
            def _():
                pltpu.sync_copy(buf_v, cnt_s.at[pl.ds(j * RB, RB)])
        plsc.subcore_barrier()
        for k in range(16):
            j = s + k * 16

            @pl.when(j < NEB)
            def _():
                pltpu.sync_copy(idx_hbm.at[pl.ds(j * EB, EB)], idx_v)
                pltpu.sync_copy(ones_v, cnt_s.at[idx_v], add=True)
        plsc.subcore_barrier()
        for k in range(2):
            j = s + k * 16

            @pl.when(j < NRB)
            def _():
                pltpu.sync_copy(cnt_s.at[pl.ds(j * RB, RB)], buf_v)
                for i in range(RB // 16):
                    v = buf_v[pl.ds(i * 16, 16)]
                    buf_v[pl.ds(i * 16, 16)] = 1.0 / jnp.maximum(v, 1.0)
                pltpu.sync_copy(buf_v, out_hbm.at[pl.ds(j * RB, RB)])

    @pl.when(c == 0)
    def _():
        run(dst_hbm, inv_p, cnt_sh)

    @pl.when(c == 1)
    def _():
        run(src_hbm, inv_u, cnt_sh)


@functools.partial(
    pl.kernel,
    out_type=jax.ShapeDtypeStruct((NPR * NCHUNK, 16), _f32),
    mesh=_mesh,
    scratch_types=[
        pltpu.VMEM((EB,), jnp.int32),
        pltpu.VMEM((EB,), jnp.int32),
        pltpu.VMEM((EB, 16), _f32),
        pltpu.VMEM((RB, 16), _f32),
        pltpu.VMEM((RB,), jnp.int32),
        pltpu.VMEM_SHARED((NPR, 16), _f32),
        pltpu.SemaphoreType.DMA,
    ],
    compiler_params=_sc_params,
)
def _sc_agg_dir(hf, gat8_hbm, sct_hbm, agg_out,
                ig_v, is_v, rw_v, zb_v, iot_v, acc, sem):
    s = lax.axis_index("s")
    c = lax.axis_index("c")
    VSZ = NU * NCHUNK - (NCHUNK - 1)

    def zrow(i, _):
        zb_v[i] = jnp.zeros((16,), _f32)
        return 0

    lax.fori_loop(0, RB, zrow, 0)

    def irow(i, _):
        iot_v[pl.ds(i * 16, 16)] = (lax.iota(jnp.int32, 16) + i * 16) * NCHUNK
        return 0

    lax.fori_loop(0, RB // 16, irow, 0)

    def one_pass(chunk, hf, gat8_hbm, sct_hbm, out_hbm):
        for k in range(2):
            j = s + k * 16

            @pl.when(j < NRB)
            def _():
                pltpu.sync_copy(zb_v, acc.at[pl.ds(j * RB, RB)])
        plsc.subcore_barrier()
        hview = hf.at[pl.ds(chunk, VSZ)]
        for k in range(16):
            j = s + k * 16

            @pl.when(j < NEB)
            def _():
                pltpu.sync_copy(gat8_hbm.at[pl.ds(j * EB, EB)], ig_v)
                pltpu.sync_copy(sct_hbm.at[pl.ds(j * EB, EB)], is_v)
                pltpu.async_copy(hview.at[ig_v], rw_v, sem).wait()
                pltpu.sync_copy(rw_v, acc.at[is_v], add=True)
        plsc.subcore_barrier()
        oview = out_hbm.at[pl.ds(chunk, VSZ)]
        for k in range(2):
            j = s + k * 16

            @pl.when(j < NRB)
            def _():
                base = j * (RB * NCHUNK)

                def wrow(i, _):
                    ig_v[pl.ds(i * 16, 16)] = iot_v[pl.ds(i * 16, 16)] + base
                    return 0

                lax.fori_loop(0, RB // 16, wrow, 0)
                pltpu.sync_copy(acc.at[pl.ds(j * RB, RB)], rw_v)
                pltpu.sync_copy(rw_v, oview.at[ig_v])
        plsc.subcore_barrier()

    for cc in range(CPS):
        chunk = c * CPS + cc
        one_pass(chunk, hf, gat8_hbm, sct_hbm, agg_out)


def _lin_body(x_ref, w_ref, b_ref, o_ref):
    o_ref[...] = lax.dot_general(
        x_ref[...], w_ref[...], (((1,), (1,)), ((), ())),
        preferred_element_type=_f32) + b_ref[...]


def _linear(x, w, b):
    n, fi = x.shape
    fo = w.shape[0]
    return pl.pallas_call(
        _lin_body,
        grid=(n // BN,),
        in_specs=[pl.BlockSpec((BN, fi), lambda i: (i, 0)),
                  pl.BlockSpec((fo, fi), lambda i: (0, 0)),
                  pl.BlockSpec((1, fo), lambda i: (0, 0))],
        out_specs=pl.BlockSpec((BN, fo), lambda i: (i, 0)),
        out_shape=jax.ShapeDtypeStruct((n, fo), _f32),
    )(x, w, b.reshape(1, fo))


def _upd_body(agg_ref, inv_ref, h_ref, wl_ref, bl_ref, wr_ref, o_ref):
    a = agg_ref[...] * inv_ref[...]
    t = lax.dot_general(a, wl_ref[...], (((1,), (1,)), ((), ())),
                        preferred_element_type=_f32)
    t = t + lax.dot_general(h_ref[...], wr_ref[...], (((1,), (1,)), ((), ())),
                            preferred_element_type=_f32)
    o_ref[...] = jnp.maximum(t + bl_ref[...], 0.0)


def _update(agg, inv, h, wl, bl, wr):
    n = h.shape[0]
    return pl.pallas_call(
        _upd_body,
        grid=(n // BN,),
        in_specs=[pl.BlockSpec((BN, H), lambda i: (i, 0)),
                  pl.BlockSpec((BN, 1), lambda i: (i, 0)),
                  pl.BlockSpec((BN, H), lambda i: (i, 0)),
                  pl.BlockSpec((H, H), lambda i: (0, 0)),
                  pl.BlockSpec((1, H), lambda i: (0, 0)),
                  pl.BlockSpec((H, H), lambda i: (0, 0))],
        out_specs=pl.BlockSpec((BN, H), lambda i: (i, 0)),
        out_shape=jax.ShapeDtypeStruct((n, H), _f32),
    )(agg, inv.reshape(n, 1), h, wl, bl.reshape(1, H), wr)


def _updo_body(agg_ref, inv_ref, h_ref, wl_ref, bl_ref, wr_ref, wo_ref,
               bo_ref, o_ref):
    a = agg_ref[...] * inv_ref[...]
    t = lax.dot_general(a, wl_ref[...], (((1,), (1,)), ((), ())),
                        preferred_element_type=_f32)
    t = t + lax.dot_general(h_ref[...], wr_ref[...], (((1,), (1,)), ((), ())),
                            preferred_element_type=_f32)
    t = jnp.maximum(t + bl_ref[...], 0.0)
    o_ref[...] = lax.dot_general(t, wo_ref[...], (((1,), (1,)), ((), ())),
                                 preferred_element_type=_f32) + bo_ref[...]


def _update_out(agg, inv, h, wl, bl, wr, wo, bo):
    n = h.shape[0]
    fo = wo.shape[0]
    return pl.pallas_call(
        _updo_body,
        grid=(n // BN,),
        in_specs=[pl.BlockSpec((BN, H), lambda i: (i, 0)),
                  pl.BlockSpec((BN, 1), lambda i: (i, 0)),
                  pl.BlockSpec((BN, H), lambda i: (i, 0)),
                  pl.BlockSpec((H, H), lambda i: (0, 0)),
                  pl.BlockSpec((1, H), lambda i: (0, 0)),
                  pl.BlockSpec((H, H), lambda i: (0, 0)),
                  pl.BlockSpec((fo, H), lambda i: (0, 0)),
                  pl.BlockSpec((1, fo), lambda i: (0, 0))],
        out_specs=pl.BlockSpec((BN, fo), lambda i: (i, 0)),
        out_shape=jax.ShapeDtypeStruct((n, fo), _f32),
    )(agg, inv.reshape(n, 1), h, wl, bl.reshape(1, H), wr, wo,
      bo.reshape(1, fo))


def kernel(x_user, x_problem, edge_index, user_W, user_b, problem_W, problem_b,
           l0_u2p_Wl, l0_u2p_bl, l0_u2p_Wr, l0_p2u_Wl, l0_p2u_bl, l0_p2u_Wr,
           l1_u2p_Wl, l1_u2p_bl, l1_u2p_Wr, l1_p2u_Wl, l1_p2u_bl, l1_p2u_Wr,
           out_user_W, out_user_b, out_problem_W, out_problem_b):
    src = edge_index[0].astype(jnp.int32)
    dst = edge_index[1].astype(jnp.int32)
    src8 = src * NCHUNK
    dst8 = dst * NCHUNK

    inv_u, inv_p = _sc_counts(src, dst)

    hu = _linear(x_user, user_W, user_b)
    hp = _linear(x_problem, problem_W, problem_b)

    aggPf = _sc_agg_dir(hu.reshape(NU * NCHUNK, 16), src8, dst)
    hp_new = _update(aggPf.reshape(NPR, H), inv_p, hp, l0_u2p_Wl, l0_u2p_bl,
                     l0_u2p_Wr)
    aggUf = _sc_agg_dir(hp.reshape(NPR * NCHUNK, 16), dst8, src)
    hu = _update(aggUf.reshape(NU, H), inv_u, hu, l0_p2u_Wl, l0_p2u_bl,
                 l0_p2u_Wr)
    hp = hp_new
    aggPf = _sc_agg_dir(hu.reshape(NU * NCHUNK, 16), src8, dst)
    out_p = _update_out(aggPf.reshape(NPR, H), inv_p, hp, l1_u2p_Wl,
                        l1_u2p_bl, l1_u2p_Wr, out_problem_W, out_problem_b)
    aggUf = _sc_agg_dir(hp.reshape(NPR * NCHUNK, 16), dst8, src)
    out_u = _update_out(aggUf.reshape(NU, H), inv_u, hu, l1_p2u_Wl,
                        l1_p2u_bl, l1_p2u_Wr, out_user_W, out_user_b)
    return (out_u, out_p)


# speedup vs baseline: 1.3527x; 1.1991x over previous
import functools

import jax
import jax.numpy as jnp
from jax import lax
from jax.experimental import pallas as pl
from jax.experimental.pallas import tpu as pltpu
from jax.experimental.pallas import tpu_sc as plsc

NU = 50000
NPR = 50000
EDG = 500000
H = 128
NCHUNK = 8
CPS = NCHUNK // 2
EB = 2000
NEB = EDG // EB
RB = 2000
NRB = NU // RB
BN = 2000
_f32 = jnp.float32

_mesh = plsc.VectorSubcoreMesh(core_axis_name="c", subcore_axis_name="s")
_sc_params = pltpu.CompilerParams(use_tc_tiling_on_sc=False)


@functools.partial(
    pl.kernel,
    out_type=[jax.ShapeDtypeStruct((NU,), _f32),
              jax.ShapeDtypeStruct((NPR,), _f32)],
    mesh=_mesh,
    scratch_types=[
        pltpu.VMEM((EB,), jnp.int32),
        pltpu.VMEM((EB,), _f32),
        pltpu.VMEM((RB,), _f32),
        pltpu.VMEM_SHARED((NU,), _f32),
    ],
    compiler_params=_sc_params,
)
def _sc_counts(src_hbm, dst_hbm, inv_u, inv_p, idx_v, ones_v, buf_v, cnt_sh):
    s = lax.axis_index("s")
    c = lax.axis_index("c")

    for i in range(EB // 16):
        ones_v[pl.ds(i * 16, 16)] = jnp.full((16,), 1.0, _f32)
    for i in range(RB // 16):
        buf_v[pl.ds(i * 16, 16)] = jnp.zeros((16,), _f32)

    def run(idx_hbm, out_hbm, cnt_s):
        for k in range(2):
            j = s + k * 16

            @pl.when(j < NRB)
            def _():
                pltpu.sync_copy(buf_v, cnt_s.at[pl.ds(j * RB, RB)])
        plsc.subcore_barrier()
        for k in range(16):
            j = s + k * 16

            @pl.when(j < NEB)
            def _():
                pltpu.sync_copy(idx_hbm.at[pl.ds(j * EB, EB)], idx_v)
                pltpu.sync_copy(ones_v, cnt_s.at[idx_v], add=True)
        plsc.subcore_barrier()
        for k in range(2):
            j = s + k * 16

            @pl.when(j < NRB)
            def _():
                pltpu.sync_copy(cnt_s.at[pl.ds(j * RB, RB)], buf_v)
                for i in range(RB // 16):
                    v = buf_v[pl.ds(i * 16, 16)]
                    buf_v[pl.ds(i * 16, 16)] = 1.0 / jnp.maximum(v, 1.0)
                pltpu.sync_copy(buf_v, out_hbm.at[pl.ds(j * RB, RB)])

    @pl.when(c == 0)
    def _():
        run(dst_hbm, inv_p, cnt_sh)

    @pl.when(c == 1)
    def _():
        run(src_hbm, inv_u, cnt_sh)


@functools.partial(
    pl.kernel,
    out_type=jax.ShapeDtypeStruct((NPR * NCHUNK, 16), _f32),
    mesh=_mesh,
    scratch_types=[
        pltpu.VMEM((2, EB), jnp.int32),
        pltpu.VMEM((2, EB), jnp.int32),
        pltpu.VMEM((EB, 16), _f32),
        pltpu.VMEM((RB, 16), _f32),
        pltpu.VMEM((RB,), jnp.int32),
        pltpu.VMEM((RB,), jnp.int32),
        pltpu.VMEM_SHARED((NPR, 16), _f32),
        pltpu.SemaphoreType.DMA,
        pltpu.SemaphoreType.DMA,
    ],
    compiler_params=_sc_params,
)
def _sc_agg_dir(hf, gat8_hbm, sct_hbm, agg_out,
                ig_v, is_v, rw_v, zb_v, iot_v, wb_v, acc, sem, semi):
    s = lax.axis_index("s")
    c = lax.axis_index("c")
    VSZ = NU * NCHUNK - (NCHUNK - 1)

    def zrow(i, _):
        zb_v[i] = jnp.zeros((16,), _f32)
        return 0

    lax.fori_loop(0, RB, zrow, 0)

    def irow(i, _):
        iot_v[pl.ds(i * 16, 16)] = (lax.iota(jnp.int32, 16) + i * 16) * NCHUNK
        return 0

    lax.fori_loop(0, RB // 16, irow, 0)

    def one_pass(chunk, hf, gat8_hbm, sct_hbm, out_hbm):
        for k in range(2):
            j = s + k * 16

            @pl.when(j < NRB)
            def _():
                pltpu.sync_copy(zb_v, acc.at[pl.ds(j * RB, RB)])
        plsc.subcore_barrier()
        hview = hf.at[pl.ds(chunk, VSZ)]

        def fetch_idx(k):
            if k >= 16:
                return
            p = k % 2
            j = s + k * 16

            @pl.when(j < NEB)
            def _():
                pltpu.async_copy(gat8_hbm.at[pl.ds(j * EB, EB)], ig_v.at[p], semi)
                pltpu.async_copy(sct_hbm.at[pl.ds(j * EB, EB)], is_v.at[p], semi)

        fetch_idx(0)
        for k in range(16):
            p = k % 2
            j = s + k * 16

            @pl.when(j < NEB)
            def _():
                pltpu.make_async_copy(gat8_hbm.at[pl.ds(j * EB, EB)],
                                      ig_v.at[p], semi).wait()
                pltpu.make_async_copy(sct_hbm.at[pl.ds(j * EB, EB)],
                                      is_v.at[p], semi).wait()
            fetch_idx(k + 1)

            @pl.when(j < NEB)
            def _():
                pltpu.async_copy(hview.at[ig_v.at[p]], rw_v, sem).wait()
                pltpu.sync_copy(rw_v, acc.at[is_v.at[p]], add=True)
        plsc.subcore_barrier()
        oview = out_hbm.at[pl.ds(chunk, VSZ)]
        for k in range(2):
            j = s + k * 16

            @pl.when(j < NRB)
            def _():
                base = j * (RB * NCHUNK)

                def wrow(i, _):
                    wb_v[pl.ds(i * 16, 16)] = iot_v[pl.ds(i * 16, 16)] + base
                    return 0

                lax.fori_loop(0, RB // 16, wrow, 0)
                pltpu.sync_copy(acc.at[pl.ds(j * RB, RB)], rw_v)
                pltpu.sync_copy(rw_v, oview.at[wb_v])
        plsc.subcore_barrier()

    for cc in range(CPS):
        chunk = c * CPS + cc
        one_pass(chunk, hf, gat8_hbm, sct_hbm, agg_out)


def _lin_body(x_ref, w_ref, b_ref, o_ref):
    o_ref[...] = lax.dot_general(
        x_ref[...], w_ref[...], (((1,), (1,)), ((), ())),
        preferred_element_type=_f32) + b_ref[...]


def _linear(x, w, b):
    n, fi = x.shape
    fo = w.shape[0]
    return pl.pallas_call(
        _lin_body,
        grid=(n // BN,),
        in_specs=[pl.BlockSpec((BN, fi), lambda i: (i, 0)),
                  pl.BlockSpec((fo, fi), lambda i: (0, 0)),
                  pl.BlockSpec((1, fo), lambda i: (0, 0))],
        out_specs=pl.BlockSpec((BN, fo), lambda i: (i, 0)),
        out_shape=jax.ShapeDtypeStruct((n, fo), _f32),
    )(x, w, b.reshape(1, fo))


def _upd_body(agg_ref, inv_ref, h_ref, wl_ref, bl_ref, wr_ref, o_ref):
    a = agg_ref[...] * inv_ref[...]
    t = lax.dot_general(a, wl_ref[...], (((1,), (1,)), ((), ())),
                        preferred_element_type=_f32)
    t = t + lax.dot_general(h_ref[...], wr_ref[...], (((1,), (1,)), ((), ())),
                            preferred_element_type=_f32)
    o_ref[...] = jnp.maximum(t + bl_ref[...], 0.0)


def _update(agg, inv, h, wl, bl, wr):
    n = h.shape[0]
    return pl.pallas_call(
        _upd_body,
        grid=(n // BN,),
        in_specs=[pl.BlockSpec((BN, H), lambda i: (i, 0)),
                  pl.BlockSpec((BN, 1), lambda i: (i, 0)),
                  pl.BlockSpec((BN, H), lambda i: (i, 0)),
                  pl.BlockSpec((H, H), lambda i: (0, 0)),
                  pl.BlockSpec((1, H), lambda i: (0, 0)),
                  pl.BlockSpec((H, H), lambda i: (0, 0))],
        out_specs=pl.BlockSpec((BN, H), lambda i: (i, 0)),
        out_shape=jax.ShapeDtypeStruct((n, H), _f32),
    )(agg, inv.reshape(n, 1), h, wl, bl.reshape(1, H), wr)


def _updo_body(agg_ref, inv_ref, h_ref, wl_ref, bl_ref, wr_ref, wo_ref,
               bo_ref, o_ref):
    a = agg_ref[...] * inv_ref[...]
    t = lax.dot_general(a, wl_ref[...], (((1,), (1,)), ((), ())),
                        preferred_element_type=_f32)
    t = t + lax.dot_general(h_ref[...], wr_ref[...], (((1,), (1,)), ((), ())),
                            preferred_element_type=_f32)
    t = jnp.maximum(t + bl_ref[...], 0.0)
    o_ref[...] = lax.dot_general(t, wo_ref[...], (((1,), (1,)), ((), ())),
                                 preferred_element_type=_f32) + bo_ref[...]


def _update_out(agg, inv, h, wl, bl, wr, wo, bo):
    n = h.shape[0]
    fo = wo.shape[0]
    return pl.pallas_call(
        _updo_body,
        grid=(n // BN,),
        in_specs=[pl.BlockSpec((BN, H), lambda i: (i, 0)),
                  pl.BlockSpec((BN, 1), lambda i: (i, 0)),
                  pl.BlockSpec((BN, H), lambda i: (i, 0)),
                  pl.BlockSpec((H, H), lambda i: (0, 0)),
                  pl.BlockSpec((1, H), lambda i: (0, 0)),
                  pl.BlockSpec((H, H), lambda i: (0, 0)),
                  pl.BlockSpec((fo, H), lambda i: (0, 0)),
                  pl.BlockSpec((1, fo), lambda i: (0, 0))],
        out_specs=pl.BlockSpec((BN, fo), lambda i: (i, 0)),
        out_shape=jax.ShapeDtypeStruct((n, fo), _f32),
    )(agg, inv.reshape(n, 1), h, wl, bl.reshape(1, H), wr, wo,
      bo.reshape(1, fo))


def kernel(x_user, x_problem, edge_index, user_W, user_b, problem_W, problem_b,
           l0_u2p_Wl, l0_u2p_bl, l0_u2p_Wr, l0_p2u_Wl, l0_p2u_bl, l0_p2u_Wr,
           l1_u2p_Wl, l1_u2p_bl, l1_u2p_Wr, l1_p2u_Wl, l1_p2u_bl, l1_p2u_Wr,
           out_user_W, out_user_b, out_problem_W, out_problem_b):
    src = edge_index[0].astype(jnp.int32)
    dst = edge_index[1].astype(jnp.int32)
    src8 = src * NCHUNK
    dst8 = dst * NCHUNK

    inv_u, inv_p = _sc_counts(src, dst)

    hu = _linear(x_user, user_W, user_b)
    hp = _linear(x_problem, problem_W, problem_b)

    aggPf = _sc_agg_dir(hu.reshape(NU * NCHUNK, 16), src8, dst)
    hp_new = _update(aggPf.reshape(NPR, H), inv_p, hp, l0_u2p_Wl, l0_u2p_bl,
                     l0_u2p_Wr)
    aggUf = _sc_agg_dir(hp.reshape(NPR * NCHUNK, 16), dst8, src)
    hu = _update(aggUf.reshape(NU, H), inv_u, hu, l0_p2u_Wl, l0_p2u_bl,
                 l0_p2u_Wr)
    hp = hp_new
    aggPf = _sc_agg_dir(hu.reshape(NU * NCHUNK, 16), src8, dst)
    out_p = _update_out(aggPf.reshape(NPR, H), inv_p, hp, l1_u2p_Wl,
                        l1_u2p_bl, l1_u2p_Wr, out_problem_W, out_problem_b)
    aggUf = _sc_agg_dir(hp.reshape(NPR * NCHUNK, 16), dst8, src)
    out_u = _update_out(aggUf.reshape(NU, H), inv_u, hu, l1_p2u_Wl,
                        l1_p2u_bl, l1_p2u_Wr, out_user_W, out_user_b)
    return (out_u, out_p)
